# Initial kernel scaffold; baseline (speedup 1.0000x reference)
import jax
import jax.numpy as jnp
from jax.experimental import pallas as pl


def kernel(x_main, x_mod, csr_idx_unit, csr_idx_view):
    raise NotImplementedError("write your pallas kernel here")


# speedup vs baseline: 88.9700x; 88.9700x over previous
import functools
import jax
import jax.numpy as jnp
from jax import lax
from jax.experimental import pallas as pl
from jax.experimental.pallas import tpu as pltpu
from jax.experimental.pallas import tpu_sc as plsc

D = 128
LANES = 16
NV = D // LANES
NC = 2
NS = 16
NW = NC * NS


def _make_stage(nrows, nseg_pad, spw, s_blk, r_chunk):
    assert nseg_pad == NW * spw and spw % s_blk == 0 and s_blk % 8 == 0
    nblk = spw // s_blk
    r_step = r_chunk - 8
    mesh = plsc.VectorSubcoreMesh(core_axis_name="c", subcore_axis_name="s")

    @functools.partial(
        pl.kernel,
        mesh=mesh,
        out_type=jax.ShapeDtypeStruct((nseg_pad, D), jnp.float32),
        scratch_types=[
            pltpu.VMEM((s_blk + LANES,), jnp.int32),
            pltpu.VMEM((r_chunk, D), jnp.float32),
            pltpu.VMEM((s_blk, D), jnp.float32),
            pltpu.SemaphoreType.DMA,
        ],
    )
    def stage(x_hbm, offs_hbm, out_hbm, offs_v, buf_v, out_v, sem):
        wid = lax.axis_index("s") * NC + lax.axis_index("c")
        neg = jnp.full((LANES,), -jnp.inf, jnp.float32)
        zeros = jnp.zeros((LANES,), jnp.float32)

        def blk_body(blk, carry):
            seg0 = wid * spw + blk * s_blk
            pltpu.async_copy(offs_hbm.at[pl.ds(seg0, s_blk + LANES)], offs_v, sem).wait()

            def seg_body(j, cst):
                v = offs_v[pl.ds(j, LANES)]
                beg, end = v[0], v[1]

                def row_body(r, rst):
                    def refill(arg):
                        r0c = (jnp.minimum(r, jnp.int32(nrows - r_chunk)) // 8) * 8
                        pltpu.async_copy(
                            x_hbm.at[pl.ds(r0c, r_chunk)], buf_v, sem
                        ).wait()
                        return (r0c, r0c + jnp.int32(r_chunk))

                    lo, hi = lax.cond(r >= rst[1], refill, lambda arg: arg, rst[:2])
                    row = r - lo
                    acc = [
                        jnp.maximum(rst[2 + i], buf_v[row, pl.ds(i * LANES, LANES)])
                        for i in range(NV)
                    ]
                    return (lo, hi, *acc)

                rst = lax.fori_loop(beg, end, row_body, tuple(cst) + (neg,) * NV)
                ne = end > beg
                for i in range(NV):
                    out_v[j, pl.ds(i * LANES, LANES)] = jnp.where(ne, rst[2 + i], zeros)
                return rst[:2]

            cst = lax.fori_loop(0, s_blk, seg_body, carry)
            pltpu.async_copy(out_v, out_hbm.at[pl.ds(seg0, s_blk)], sem).wait()
            return cst

        lax.fori_loop(0, nblk, blk_body, (jnp.int32(0), jnp.int32(0)))

    return stage


_stage1 = _make_stage(nrows=320000, nseg_pad=64000, spw=2000, s_blk=400, r_chunk=128)
_stage2 = _make_stage(nrows=64000, nseg_pad=10240, spw=320, s_blk=320, r_chunk=128)


def kernel(x_main, x_mod, csr_idx_unit, csr_idx_view):
    del x_main
    offs1 = jnp.concatenate(
        [csr_idx_unit, jnp.broadcast_to(csr_idx_unit[-1:], (23,))]
    )
    offs2 = jnp.concatenate(
        [csr_idx_view, jnp.full((10256 - 10001,), 64000, jnp.int32)]
    )
    x_agg = _stage1(x_mod, offs1)
    out = _stage2(x_agg, offs2)
    return out[:10000]


# speedup vs baseline: 128.5572x; 1.4450x over previous
import functools
import jax
import jax.numpy as jnp
from jax import lax
from jax.experimental import pallas as pl
from jax.experimental.pallas import tpu as pltpu
from jax.experimental.pallas import tpu_sc as plsc

D = 128
LANES = 16
NV = D // LANES
NC = 2
NS = 16
NW = NC * NS


def _make_stage(nrows, nseg_pad, spw, s_blk, r_chunk):
    assert nseg_pad == NW * spw and spw % s_blk == 0 and s_blk % 8 == 0
    assert nrows % r_chunk == 0 and (r_chunk & (r_chunk - 1)) == 0
    nblk = spw // s_blk
    mesh = plsc.VectorSubcoreMesh(core_axis_name="c", subcore_axis_name="s")

    @functools.partial(
        pl.kernel,
        mesh=mesh,
        out_type=jax.ShapeDtypeStruct((nseg_pad, D), jnp.float32),
        scratch_types=[
            pltpu.VMEM((s_blk + LANES,), jnp.int32),
            pltpu.VMEM((2 * r_chunk, D), jnp.float32),
            pltpu.VMEM((s_blk, D), jnp.float32),
            pltpu.SemaphoreType.DMA,
            pltpu.SemaphoreType.DMA,
            pltpu.SemaphoreType.DMA,
        ],
    )
    def stage(x_hbm, offs_hbm, out_hbm, offs_v, buf_v, out_v, sem, sem_a, sem_b):
        wid = lax.axis_index("s") * NC + lax.axis_index("c")
        neg = jnp.full((LANES,), -jnp.inf, jnp.float32)
        zeros = jnp.zeros((LANES,), jnp.float32)
        half_sems = (sem_a, sem_b)
        cint = jnp.int32(r_chunk)

        def issue_half(base, p):
            base = pl.multiple_of(base, r_chunk)
            pltpu.async_copy(
                x_hbm.at[pl.ds(base, r_chunk)],
                buf_v.at[pl.ds(p * r_chunk, r_chunk)],
                half_sems[p],
            )

        def wait_half(p):
            pltpu.make_async_copy(
                x_hbm.at[pl.ds(0, r_chunk)],
                buf_v.at[pl.ds(p * r_chunk, r_chunk)],
                half_sems[p],
            ).wait()

        def by_parity(base, fn):
            p_is0 = ((base // cint) & 1) == 0

            @pl.when(p_is0)
            def _():
                fn(0)

            @pl.when(jnp.logical_not(p_is0))
            def _():
                fn(1)

        def refill(r):
            by_parity(r, wait_half)
            nb = r + cint

            @pl.when(nb < jnp.int32(nrows))
            def _():
                by_parity(nb, lambda p: issue_half(nb, p))

        pltpu.async_copy(
            offs_hbm.at[pl.ds(wid * spw, LANES)], offs_v.at[pl.ds(0, LANES)], sem
        ).wait()
        rbeg = offs_v[pl.ds(0, LANES)][0]
        b0 = jnp.minimum((rbeg // cint) * cint, jnp.int32(nrows - r_chunk))
        by_parity(b0, lambda p: issue_half(b0, p))
        by_parity(b0, wait_half)
        hi0 = b0 + cint

        @pl.when(hi0 < jnp.int32(nrows))
        def _():
            by_parity(hi0, lambda p: issue_half(hi0, p))

        def blk_body(blk, hi_c):
            seg0 = wid * spw + blk * s_blk
            pltpu.async_copy(offs_hbm.at[pl.ds(seg0, s_blk + LANES)], offs_v, sem).wait()

            def seg_body(j, hi_s):
                v = offs_v[pl.ds(j, LANES)]
                beg, end = v[0], v[1]

                def row_body(r, rst):
                    @pl.when(r >= rst[0])
                    def _():
                        refill(r)

                    hi = jnp.where(r >= rst[0], r + cint, rst[0])
                    row = jnp.bitwise_and(r, jnp.int32(2 * r_chunk - 1))
                    acc = [
                        jnp.maximum(rst[1 + i], buf_v[row, pl.ds(i * LANES, LANES)])
                        for i in range(NV)
                    ]
                    return (hi, *acc)

                rst = lax.fori_loop(beg, end, row_body, (hi_s,) + (neg,) * NV)
                ne = end > beg
                for i in range(NV):
                    out_v[j, pl.ds(i * LANES, LANES)] = jnp.where(ne, rst[1 + i], zeros)
                return rst[0]

            hi_c = lax.fori_loop(0, s_blk, seg_body, hi_c)
            pltpu.async_copy(out_v, out_hbm.at[pl.ds(seg0, s_blk)], sem).wait()
            return hi_c

        hi_end = lax.fori_loop(0, nblk, blk_body, hi0)

        @pl.when(hi_end < jnp.int32(nrows))
        def _():
            by_parity(hi_end, wait_half)

    return stage


_stage1 = _make_stage(nrows=320000, nseg_pad=64000, spw=2000, s_blk=400, r_chunk=128)
_stage2 = _make_stage(nrows=64000, nseg_pad=10240, spw=320, s_blk=320, r_chunk=128)


def kernel(x_main, x_mod, csr_idx_unit, csr_idx_view):
    del x_main
    offs1 = jnp.concatenate(
        [csr_idx_unit, jnp.broadcast_to(csr_idx_unit[-1:], (23,))]
    )
    offs2 = jnp.concatenate(
        [csr_idx_view, jnp.full((10256 - 10001,), 64000, jnp.int32)]
    )
    x_agg = _stage1(x_mod, offs1)
    out = _stage2(x_agg, offs2)
    return out[:10000]


# speedup vs baseline: 157.1915x; 1.2227x over previous
import functools
import jax
import jax.numpy as jnp
from jax import lax
from jax.experimental import pallas as pl
from jax.experimental.pallas import tpu as pltpu
from jax.experimental.pallas import tpu_sc as plsc

D = 128
LANES = 16
NV = D // LANES
NC = 2
NS = 16
NW = NC * NS


def _make_stage(nrows, nseg_pad, spw, s_blk, r_chunk):
    assert nseg_pad == NW * spw and spw % s_blk == 0 and s_blk % 8 == 0
    assert nrows % r_chunk == 0 and (r_chunk & (r_chunk - 1)) == 0
    nblk = spw // s_blk
    mesh = plsc.VectorSubcoreMesh(core_axis_name="c", subcore_axis_name="s")

    @functools.partial(
        pl.kernel,
        mesh=mesh,
        out_type=jax.ShapeDtypeStruct((nseg_pad, D), jnp.float32),
        scratch_types=[
            pltpu.VMEM((s_blk + LANES,), jnp.int32),
            pltpu.VMEM((2 * r_chunk, D), jnp.float32),
            pltpu.VMEM((s_blk, D), jnp.float32),
            pltpu.SemaphoreType.DMA,
            pltpu.SemaphoreType.DMA,
            pltpu.SemaphoreType.DMA,
        ],
    )
    def stage(x_hbm, offs_hbm, out_hbm, offs_v, buf_v, out_v, sem, sem_a, sem_b):
        wid = lax.axis_index("s") * NC + lax.axis_index("c")
        neg = jnp.full((LANES,), -jnp.inf, jnp.float32)
        zeros = jnp.zeros((LANES,), jnp.float32)
        half_sems = (sem_a, sem_b)
        cint = jnp.int32(r_chunk)

        def issue_half(base, p):
            base = pl.multiple_of(base, r_chunk)
            pltpu.async_copy(
                x_hbm.at[pl.ds(base, r_chunk)],
                buf_v.at[pl.ds(p * r_chunk, r_chunk)],
                half_sems[p],
            )

        def wait_half(p):
            pltpu.make_async_copy(
                x_hbm.at[pl.ds(0, r_chunk)],
                buf_v.at[pl.ds(p * r_chunk, r_chunk)],
                half_sems[p],
            ).wait()

        def by_parity(base, fn):
            p_is0 = ((base // cint) & 1) == 0

            @pl.when(p_is0)
            def _():
                fn(0)

            @pl.when(jnp.logical_not(p_is0))
            def _():
                fn(1)

        def refill(r):
            by_parity(r, wait_half)
            nb = r + cint

            @pl.when(nb < jnp.int32(nrows))
            def _():
                by_parity(nb, lambda p: issue_half(nb, p))

        pltpu.async_copy(
            offs_hbm.at[pl.ds(wid * spw, LANES)], offs_v.at[pl.ds(0, LANES)], sem
        ).wait()
        rbeg = offs_v[pl.ds(0, LANES)][0]
        b0 = jnp.minimum((rbeg // cint) * cint, jnp.int32(nrows - r_chunk))
        by_parity(b0, lambda p: issue_half(b0, p))
        by_parity(b0, wait_half)
        hi0 = b0 + cint

        @pl.when(hi0 < jnp.int32(nrows))
        def _():
            by_parity(hi0, lambda p: issue_half(hi0, p))

        ring_mask = jnp.int32(2 * r_chunk - 1)

        def blk_body(blk, carry):
            seg0 = wid * spw + blk * s_blk
            pltpu.async_copy(offs_hbm.at[pl.ds(seg0, s_blk + LANES)], offs_v, sem).wait()

            def seg_body(j, cst):
                hi_s, beg = cst
                end = offs_v[pl.ds(j, LANES)][1]

                ne = end > beg

                def store(acc):
                    for i in range(NV):
                        out_v[j, pl.ds(i * LANES, LANES)] = jnp.where(
                            ne, acc[i], zeros
                        )

                def fast(_):
                    base = jnp.bitwise_and(beg, ring_mask)

                    def fbody(k, acc):
                        return tuple(
                            jnp.maximum(acc[i], buf_v[k, pl.ds(i * LANES, LANES)])
                            for i in range(NV)
                        )

                    acc = lax.fori_loop(base, base + (end - beg), fbody, (neg,) * NV)
                    store(acc)
                    return hi_s

                def slow(_):
                    def row_body(r, rst):
                        @pl.when(r >= rst[0])
                        def _():
                            refill(r)

                        hi = jnp.where(r >= rst[0], r + cint, rst[0])
                        row = jnp.bitwise_and(r, ring_mask)
                        acc = [
                            jnp.maximum(rst[1 + i], buf_v[row, pl.ds(i * LANES, LANES)])
                            for i in range(NV)
                        ]
                        return (hi, *acc)

                    rst = lax.fori_loop(beg, end, row_body, (hi_s,) + (neg,) * NV)
                    store(list(rst[1:]))
                    return rst[0]

                hi_n = lax.cond(end <= hi_s, fast, slow, 0)
                return (hi_n, end)

            carry = lax.fori_loop(0, s_blk, seg_body, carry)
            pltpu.async_copy(out_v, out_hbm.at[pl.ds(seg0, s_blk)], sem).wait()
            return carry

        hi_end, _ = lax.fori_loop(0, nblk, blk_body, (hi0, rbeg))

        @pl.when(hi_end < jnp.int32(nrows))
        def _():
            by_parity(hi_end, wait_half)

    return stage


_stage1 = _make_stage(nrows=320000, nseg_pad=64000, spw=2000, s_blk=400, r_chunk=128)
_stage2 = _make_stage(nrows=64000, nseg_pad=10240, spw=320, s_blk=320, r_chunk=128)


def kernel(x_main, x_mod, csr_idx_unit, csr_idx_view):
    del x_main
    offs1 = jnp.concatenate(
        [csr_idx_unit, jnp.broadcast_to(csr_idx_unit[-1:], (23,))]
    )
    offs2 = jnp.concatenate(
        [csr_idx_view, jnp.full((10256 - 10001,), 64000, jnp.int32)]
    )
    x_agg = _stage1(x_mod, offs1)
    out = _stage2(x_agg, offs2)
    return out[:10000]


# speedup vs baseline: 159.5101x; 1.0148x over previous
import functools
import jax
import jax.numpy as jnp
from jax import lax
from jax.experimental import pallas as pl
from jax.experimental.pallas import tpu as pltpu
from jax.experimental.pallas import tpu_sc as plsc

D = 128
LANES = 16
NV = D // LANES
NC = 2
NS = 16
NW = NC * NS


def _make_stage(nrows, nseg_pad, spw, s_blk, r_chunk):
    assert nseg_pad == NW * spw and spw % s_blk == 0 and s_blk % 8 == 0
    assert nrows % r_chunk == 0 and (r_chunk & (r_chunk - 1)) == 0
    nblk = spw // s_blk
    mesh = plsc.VectorSubcoreMesh(core_axis_name="c", subcore_axis_name="s")

    @functools.partial(
        pl.kernel,
        mesh=mesh,
        out_type=jax.ShapeDtypeStruct((nseg_pad, D), jnp.float32),
        scratch_types=[
            pltpu.VMEM((s_blk + LANES,), jnp.int32),
            pltpu.VMEM((2 * r_chunk, D), jnp.float32),
            pltpu.VMEM((s_blk, D), jnp.float32),
            pltpu.SemaphoreType.DMA,
            pltpu.SemaphoreType.DMA,
            pltpu.SemaphoreType.DMA,
        ],
    )
    def stage(x_hbm, offs_hbm, out_hbm, offs_v, buf_v, out_v, sem, sem_a, sem_b):
        wid = lax.axis_index("s") * NC + lax.axis_index("c")
        neg = jnp.full((LANES,), -jnp.inf, jnp.float32)
        zeros = jnp.zeros((LANES,), jnp.float32)
        half_sems = (sem_a, sem_b)
        cint = jnp.int32(r_chunk)

        def issue_half(base, p):
            base = pl.multiple_of(base, r_chunk)
            pltpu.async_copy(
                x_hbm.at[pl.ds(base, r_chunk)],
                buf_v.at[pl.ds(p * r_chunk, r_chunk)],
                half_sems[p],
            )

        def wait_half(p):
            pltpu.make_async_copy(
                x_hbm.at[pl.ds(0, r_chunk)],
                buf_v.at[pl.ds(p * r_chunk, r_chunk)],
                half_sems[p],
            ).wait()

        def by_parity(base, fn):
            p_is0 = ((base // cint) & 1) == 0

            @pl.when(p_is0)
            def _():
                fn(0)

            @pl.when(jnp.logical_not(p_is0))
            def _():
                fn(1)

        def refill(r):
            by_parity(r, wait_half)
            nb = r + cint

            @pl.when(nb < jnp.int32(nrows))
            def _():
                by_parity(nb, lambda p: issue_half(nb, p))

        pltpu.async_copy(
            offs_hbm.at[pl.ds(wid * spw, LANES)], offs_v.at[pl.ds(0, LANES)], sem
        ).wait()
        rbeg = offs_v[pl.ds(0, LANES)][0]
        b0 = jnp.minimum((rbeg // cint) * cint, jnp.int32(nrows - r_chunk))
        by_parity(b0, lambda p: issue_half(b0, p))
        by_parity(b0, wait_half)
        hi0 = b0 + cint

        @pl.when(hi0 < jnp.int32(nrows))
        def _():
            by_parity(hi0, lambda p: issue_half(hi0, p))

        ring_mask = jnp.int32(2 * r_chunk - 1)

        def blk_body(blk, carry):
            seg0 = wid * spw + blk * s_blk
            pltpu.async_copy(offs_hbm.at[pl.ds(seg0, s_blk + LANES)], offs_v, sem).wait()

            def seg_body(j, cst):
                hi_s, beg = cst
                end = offs_v[pl.ds(j, LANES)][1]

                ne = end > beg

                def store(acc):
                    for i in range(NV):
                        out_v[j, pl.ds(i * LANES, LANES)] = jnp.where(
                            ne, acc[i], zeros
                        )

                def fast(_):
                    base = jnp.bitwise_and(beg, ring_mask)

                    def fbody(k, acc):
                        return tuple(
                            jnp.maximum(acc[i], buf_v[k, pl.ds(i * LANES, LANES)])
                            for i in range(NV)
                        )

                    acc = lax.fori_loop(base, base + (end - beg), fbody, (neg,) * NV)
                    store(acc)
                    return hi_s

                def slow(_):
                    def row_body(r, rst):
                        @pl.when(r >= rst[0])
                        def _():
                            refill(r)

                        hi = jnp.where(r >= rst[0], r + cint, rst[0])
                        row = jnp.bitwise_and(r, ring_mask)
                        acc = [
                            jnp.maximum(rst[1 + i], buf_v[row, pl.ds(i * LANES, LANES)])
                            for i in range(NV)
                        ]
                        return (hi, *acc)

                    rst = lax.fori_loop(beg, end, row_body, (hi_s,) + (neg,) * NV)
                    store(list(rst[1:]))
                    return rst[0]

                hi_n = lax.cond(end <= hi_s, fast, slow, 0)
                return (hi_n, end)

            carry = lax.fori_loop(0, s_blk, seg_body, carry)
            pltpu.async_copy(out_v, out_hbm.at[pl.ds(seg0, s_blk)], sem).wait()
            return carry

        hi_end, _ = lax.fori_loop(0, nblk, blk_body, (hi0, rbeg))

        @pl.when(hi_end < jnp.int32(nrows))
        def _():
            by_parity(hi_end, wait_half)

    return stage


_stage1 = _make_stage(nrows=320000, nseg_pad=64000, spw=2000, s_blk=400, r_chunk=256)
_stage2 = _make_stage(nrows=64000, nseg_pad=10240, spw=320, s_blk=320, r_chunk=256)


def kernel(x_main, x_mod, csr_idx_unit, csr_idx_view):
    del x_main
    offs1 = jnp.concatenate(
        [csr_idx_unit, jnp.broadcast_to(csr_idx_unit[-1:], (23,))]
    )
    offs2 = jnp.concatenate(
        [csr_idx_view, jnp.full((10256 - 10001,), 64000, jnp.int32)]
    )
    x_agg = _stage1(x_mod, offs1)
    out = _stage2(x_agg, offs2)
    return out[:10000]


# speedup vs baseline: 160.5980x; 1.0068x over previous
import functools
import jax
import jax.numpy as jnp
from jax import lax
from jax.experimental import pallas as pl
from jax.experimental.pallas import tpu as pltpu
from jax.experimental.pallas import tpu_sc as plsc

D = 128
LANES = 16
NV = D // LANES
NC = 2
NS = 16
NW = NC * NS


def _make_stage(nrows, nseg_pad, spw, s_blk, r_chunk):
    assert nseg_pad == NW * spw and spw % s_blk == 0 and s_blk % 8 == 0
    assert nrows % r_chunk == 0 and (r_chunk & (r_chunk - 1)) == 0
    nblk = spw // s_blk
    mesh = plsc.VectorSubcoreMesh(core_axis_name="c", subcore_axis_name="s")

    @functools.partial(
        pl.kernel,
        mesh=mesh,
        out_type=jax.ShapeDtypeStruct((nseg_pad, D), jnp.float32),
        scratch_types=[
            pltpu.VMEM((s_blk + LANES,), jnp.int32),
            pltpu.VMEM((2 * r_chunk, D), jnp.float32),
            pltpu.VMEM((s_blk, D), jnp.float32),
            pltpu.SemaphoreType.DMA,
            pltpu.SemaphoreType.DMA,
            pltpu.SemaphoreType.DMA,
        ],
    )
    def stage(x_hbm, offs_hbm, out_hbm, offs_v, buf_v, out_v, sem, sem_a, sem_b):
        wid = lax.axis_index("s") * NC + lax.axis_index("c")
        neg = jnp.full((LANES,), -jnp.inf, jnp.float32)
        zeros = jnp.zeros((LANES,), jnp.float32)
        half_sems = (sem_a, sem_b)
        cint = jnp.int32(r_chunk)

        def issue_half(base, p):
            base = pl.multiple_of(base, r_chunk)
            pltpu.async_copy(
                x_hbm.at[pl.ds(base, r_chunk)],
                buf_v.at[pl.ds(p * r_chunk, r_chunk)],
                half_sems[p],
            )

        def wait_half(p):
            pltpu.make_async_copy(
                x_hbm.at[pl.ds(0, r_chunk)],
                buf_v.at[pl.ds(p * r_chunk, r_chunk)],
                half_sems[p],
            ).wait()

        def by_parity(base, fn):
            p_is0 = ((base // cint) & 1) == 0

            @pl.when(p_is0)
            def _():
                fn(0)

            @pl.when(jnp.logical_not(p_is0))
            def _():
                fn(1)

        def refill(r):
            by_parity(r, wait_half)
            nb = r + cint

            @pl.when(nb < jnp.int32(nrows))
            def _():
                by_parity(nb, lambda p: issue_half(nb, p))

        pltpu.async_copy(
            offs_hbm.at[pl.ds(wid * spw, LANES)], offs_v.at[pl.ds(0, LANES)], sem
        ).wait()
        rbeg = offs_v[pl.ds(0, LANES)][0]
        b0 = jnp.minimum((rbeg // cint) * cint, jnp.int32(nrows - r_chunk))
        by_parity(b0, lambda p: issue_half(b0, p))
        by_parity(b0, wait_half)
        hi0 = b0 + cint

        @pl.when(hi0 < jnp.int32(nrows))
        def _():
            by_parity(hi0, lambda p: issue_half(hi0, p))

        ring_mask = jnp.int32(2 * r_chunk - 1)

        def blk_body(blk, carry):
            seg0 = wid * spw + blk * s_blk
            pltpu.async_copy(offs_hbm.at[pl.ds(seg0, s_blk + LANES)], offs_v, sem).wait()

            def seg_body(j, cst):
                hi_s, beg = cst
                end = offs_v[pl.ds(j, LANES)][1]

                ne = end > beg

                def store(acc):
                    for i in range(NV):
                        out_v[j, pl.ds(i * LANES, LANES)] = jnp.where(
                            ne, acc[i], zeros
                        )

                def fast(_):
                    base = jnp.bitwise_and(beg, ring_mask)
                    lim = base + (end - beg)

                    @pl.loop(base, lim, init_carry=(neg,) * NV, step=2)
                    def facc(k, acc):
                        k2 = jnp.minimum(k + 1, lim - 1)
                        a = [
                            jnp.maximum(acc[i], buf_v[k, pl.ds(i * LANES, LANES)])
                            for i in range(NV)
                        ]
                        return tuple(
                            jnp.maximum(a[i], buf_v[k2, pl.ds(i * LANES, LANES)])
                            for i in range(NV)
                        )

                    store(facc)
                    return hi_s

                def slow(_):
                    def row_body(r, rst):
                        @pl.when(r >= rst[0])
                        def _():
                            refill(r)

                        hi = jnp.where(r >= rst[0], r + cint, rst[0])
                        row = jnp.bitwise_and(r, ring_mask)
                        acc = [
                            jnp.maximum(rst[1 + i], buf_v[row, pl.ds(i * LANES, LANES)])
                            for i in range(NV)
                        ]
                        return (hi, *acc)

                    rst = lax.fori_loop(beg, end, row_body, (hi_s,) + (neg,) * NV)
                    store(list(rst[1:]))
                    return rst[0]

                hi_n = lax.cond(end <= hi_s, fast, slow, 0)
                return (hi_n, end)

            carry = lax.fori_loop(0, s_blk, seg_body, carry)
            pltpu.async_copy(out_v, out_hbm.at[pl.ds(seg0, s_blk)], sem).wait()
            return carry

        hi_end, _ = lax.fori_loop(0, nblk, blk_body, (hi0, rbeg))

        @pl.when(hi_end < jnp.int32(nrows))
        def _():
            by_parity(hi_end, wait_half)

    return stage


_stage1 = _make_stage(nrows=320000, nseg_pad=64000, spw=2000, s_blk=400, r_chunk=256)
_stage2 = _make_stage(nrows=64000, nseg_pad=10240, spw=320, s_blk=320, r_chunk=256)


def kernel(x_main, x_mod, csr_idx_unit, csr_idx_view):
    del x_main
    offs1 = jnp.concatenate(
        [csr_idx_unit, jnp.broadcast_to(csr_idx_unit[-1:], (23,))]
    )
    offs2 = jnp.concatenate(
        [csr_idx_view, jnp.full((10256 - 10001,), 64000, jnp.int32)]
    )
    x_agg = _stage1(x_mod, offs1)
    out = _stage2(x_agg, offs2)
    return out[:10000]
